# baseline (device time: 247129 ns/iter reference)
import jax
import jax.numpy as jnp
from jax import lax
from jax.experimental import pallas as pl
from jax.experimental.pallas import tpu as pltpu

N_DEV = 32


def kernel(x, Wg, Wu, Wd):
    m, d = x.shape

    def body(x_ref, wg_ref, wu_ref, wd_ref, out_ref,
             comm_ref, send_sems, recv_sems, credit_sem):
        my = lax.axis_index("i")
        left = lax.rem(my + N_DEV - 1, N_DEV)
        right = lax.rem(my + 1, N_DEV)

        barrier_sem = pltpu.get_barrier_semaphore()
        for nbr in (left, right):
            pl.semaphore_signal(
                barrier_sem, inc=1,
                device_id=(nbr,), device_id_type=pl.DeviceIdType.MESH,
            )
        pl.semaphore_wait(barrier_sem, 2)

        xb = x_ref[...].astype(jnp.bfloat16)
        gate = jnp.dot(xb, wg_ref[...].astype(jnp.bfloat16),
                       preferred_element_type=jnp.float32)
        up = jnp.dot(xb, wu_ref[...].astype(jnp.bfloat16),
                     preferred_element_type=jnp.float32)
        hidden = gate * (up * jax.nn.sigmoid(up))
        partial = jnp.dot(hidden.astype(jnp.bfloat16),
                          wd_ref[...].astype(jnp.bfloat16),
                          preferred_element_type=jnp.float32)

        out_ref[...] = partial
        comm_ref[0, :, :] = partial.astype(jnp.bfloat16)

        for h in range(N_DEV - 1):
            s_send = h % 2
            s_recv = (h + 1) % 2
            if h >= 2:
                pl.semaphore_wait(credit_sem, 1)
            rdma = pltpu.make_async_remote_copy(
                src_ref=comm_ref.at[s_send],
                dst_ref=comm_ref.at[s_recv],
                send_sem=send_sems.at[s_send],
                recv_sem=recv_sems.at[s_recv],
                device_id=(right,),
                device_id_type=pl.DeviceIdType.MESH,
            )
            rdma.start()
            rdma.wait()
            out_ref[...] = out_ref[...] + comm_ref[s_recv, :, :].astype(jnp.float32)
            if h <= N_DEV - 4:
                pl.semaphore_signal(
                    credit_sem, inc=1,
                    device_id=(left,), device_id_type=pl.DeviceIdType.MESH,
                )

    return pl.pallas_call(
        body,
        out_shape=jax.ShapeDtypeStruct((m, d), jnp.float32),
        in_specs=[pl.BlockSpec(memory_space=pltpu.VMEM)] * 4,
        out_specs=pl.BlockSpec(memory_space=pltpu.VMEM),
        scratch_shapes=[
            pltpu.VMEM((2, m, d), jnp.bfloat16),
            pltpu.SemaphoreType.DMA((2,)),
            pltpu.SemaphoreType.DMA((2,)),
            pltpu.SemaphoreType.REGULAR,
        ],
        compiler_params=pltpu.CompilerParams(collective_id=0),
    )(x, Wg, Wu, Wd)


# device time: 48805 ns/iter; 5.0636x vs baseline; 5.0636x over previous
import jax
import jax.numpy as jnp
from jax import lax
from jax.experimental import pallas as pl
from jax.experimental.pallas import tpu as pltpu

N_DEV = 32
M = 512

RS_ROWS = [M >> (k + 1) for k in range(5)]
AG_ROWS = [16 << j for j in range(5)]
_offs, _o = [], 0
for _r in RS_ROWS + AG_ROWS:
    _offs.append(_o)
    _o += _r
STAGE_ROWS = _o


def kernel(x, Wg, Wu, Wd):
    m, d = x.shape

    def body(x_ref, wg_ref, wu_ref, wd_ref, out_ref,
             send_ref, recv_ref, send_sems, recv_sems):
        p = lax.axis_index("i")

        barrier_sem = pltpu.get_barrier_semaphore()
        for s in (16, 8, 4, 2, 1):
            pl.semaphore_signal(
                barrier_sem, inc=1,
                device_id=(p ^ s,), device_id_type=pl.DeviceIdType.MESH,
            )
        pl.semaphore_wait(barrier_sem, 5)

        xb = x_ref[...].astype(jnp.bfloat16)
        gate = jnp.dot(xb, wg_ref[...].astype(jnp.bfloat16),
                       preferred_element_type=jnp.float32)
        up = jnp.dot(xb, wu_ref[...].astype(jnp.bfloat16),
                     preferred_element_type=jnp.float32)
        hidden = gate * (up * jax.nn.sigmoid(up))
        out_ref[...] = jnp.dot(hidden.astype(jnp.bfloat16),
                               wd_ref[...].astype(jnp.bfloat16),
                               preferred_element_type=jnp.float32)

        for k in range(5):
            half = RS_ROWS[k]
            off = _offs[k]
            bit = (p >> (4 - k)) & 1
            seg_start = (p >> (5 - k)) * (M >> k)
            keep_start = seg_start + bit * half
            send_start = seg_start + (1 - bit) * half

            send_ref[pl.ds(off, half), :] = (
                out_ref[pl.ds(send_start, half), :].astype(jnp.bfloat16))
            rdma = pltpu.make_async_remote_copy(
                src_ref=send_ref.at[pl.ds(off, half)],
                dst_ref=recv_ref.at[pl.ds(off, half)],
                send_sem=send_sems.at[k],
                recv_sem=recv_sems.at[k],
                device_id=(p ^ (16 >> k),),
                device_id_type=pl.DeviceIdType.MESH,
            )
            rdma.start()
            rdma.wait()
            out_ref[pl.ds(keep_start, half), :] = (
                out_ref[pl.ds(keep_start, half), :]
                + recv_ref[pl.ds(off, half), :].astype(jnp.float32))

        for j in range(5):
            b = AG_ROWS[j]
            off = _offs[5 + j]
            bit = (p >> j) & 1
            my_start = (p >> j) * (16 << j)
            merged_start = (p >> (j + 1)) * (32 << j)
            partner_start = merged_start + (1 - bit) * b

            send_ref[pl.ds(off, b), :] = (
                out_ref[pl.ds(my_start, b), :].astype(jnp.bfloat16))
            rdma = pltpu.make_async_remote_copy(
                src_ref=send_ref.at[pl.ds(off, b)],
                dst_ref=recv_ref.at[pl.ds(off, b)],
                send_sem=send_sems.at[5 + j],
                recv_sem=recv_sems.at[5 + j],
                device_id=(p ^ (1 << j),),
                device_id_type=pl.DeviceIdType.MESH,
            )
            rdma.start()
            rdma.wait()
            out_ref[pl.ds(partner_start, b), :] = (
                recv_ref[pl.ds(off, b), :].astype(jnp.float32))

    return pl.pallas_call(
        body,
        out_shape=jax.ShapeDtypeStruct((m, d), jnp.float32),
        in_specs=[pl.BlockSpec(memory_space=pltpu.VMEM)] * 4,
        out_specs=pl.BlockSpec(memory_space=pltpu.VMEM),
        scratch_shapes=[
            pltpu.VMEM((STAGE_ROWS, d), jnp.bfloat16),
            pltpu.VMEM((STAGE_ROWS, d), jnp.bfloat16),
            pltpu.SemaphoreType.DMA((10,)),
            pltpu.SemaphoreType.DMA((10,)),
        ],
        compiler_params=pltpu.CompilerParams(collective_id=0),
    )(x, Wg, Wu, Wd)


# device time: 32617 ns/iter; 7.5767x vs baseline; 1.4963x over previous
import jax
import jax.numpy as jnp
from jax import lax
from jax.experimental import pallas as pl
from jax.experimental.pallas import tpu as pltpu

N_DEV = 32
CHUNK = 512 // N_DEV


def kernel(x, Wg, Wu, Wd):
    m, d = x.shape

    def body(x_ref, wg_ref, wu_ref, wd_ref, out_ref,
             send_all, rs_recv, ag_recv,
             rs_send_sem, rs_recv_sem, ag_send_sem, ag_recv_sem):
        p = lax.axis_index("i")

        barrier_sem = pltpu.get_barrier_semaphore()
        for o in range(1, N_DEV):
            pl.semaphore_signal(
                barrier_sem, inc=1,
                device_id=(lax.rem(p + o, N_DEV),),
                device_id_type=pl.DeviceIdType.MESH,
            )
        pl.semaphore_wait(barrier_sem, N_DEV - 1)

        xb = x_ref[...].astype(jnp.bfloat16)
        gate = jnp.dot(xb, wg_ref[...].astype(jnp.bfloat16),
                       preferred_element_type=jnp.float32)
        up = jnp.dot(xb, wu_ref[...].astype(jnp.bfloat16),
                     preferred_element_type=jnp.float32)
        hidden = gate * (up * jax.nn.sigmoid(up))
        partial = jnp.dot(hidden.astype(jnp.bfloat16),
                          wd_ref[...].astype(jnp.bfloat16),
                          preferred_element_type=jnp.float32)
        send_all[...] = partial.astype(jnp.bfloat16)

        for o in range(1, N_DEV):
            t = lax.rem(p + o, N_DEV)
            rs_rdma = pltpu.make_async_remote_copy(
                src_ref=send_all.at[pl.ds(t * CHUNK, CHUNK)],
                dst_ref=rs_recv.at[pl.ds(p * CHUNK, CHUNK)],
                send_sem=rs_send_sem,
                recv_sem=rs_recv_sem,
                device_id=(t,),
                device_id_type=pl.DeviceIdType.MESH,
            )
            rs_rdma.start()

        rs_recv[pl.ds(p * CHUNK, CHUNK), :] = send_all[pl.ds(p * CHUNK, CHUNK), :]
        for _ in range(N_DEV - 1):
            rs_rdma.wait_recv()

        red = rs_recv[pl.ds(0, CHUNK), :].astype(jnp.float32)
        for q in range(1, N_DEV):
            red = red + rs_recv[pl.ds(q * CHUNK, CHUNK), :].astype(jnp.float32)

        ag_recv[pl.ds(p * CHUNK, CHUNK), :] = red.astype(jnp.bfloat16)
        for o in range(1, N_DEV):
            t = lax.rem(p + o, N_DEV)
            ag_rdma = pltpu.make_async_remote_copy(
                src_ref=ag_recv.at[pl.ds(p * CHUNK, CHUNK)],
                dst_ref=ag_recv.at[pl.ds(p * CHUNK, CHUNK)],
                send_sem=ag_send_sem,
                recv_sem=ag_recv_sem,
                device_id=(t,),
                device_id_type=pl.DeviceIdType.MESH,
            )
            ag_rdma.start()
        for _ in range(N_DEV - 1):
            ag_rdma.wait_recv()

        out_ref[...] = ag_recv[...].astype(jnp.float32)

        for _ in range(N_DEV - 1):
            rs_rdma.wait_send()
            ag_rdma.wait_send()

    return pl.pallas_call(
        body,
        out_shape=jax.ShapeDtypeStruct((m, d), jnp.float32),
        in_specs=[pl.BlockSpec(memory_space=pltpu.VMEM)] * 4,
        out_specs=pl.BlockSpec(memory_space=pltpu.VMEM),
        scratch_shapes=[
            pltpu.VMEM((m, d), jnp.bfloat16),
            pltpu.VMEM((m, d), jnp.bfloat16),
            pltpu.VMEM((m, d), jnp.bfloat16),
            pltpu.SemaphoreType.DMA,
            pltpu.SemaphoreType.DMA,
            pltpu.SemaphoreType.DMA,
            pltpu.SemaphoreType.DMA,
        ],
        compiler_params=pltpu.CompilerParams(collective_id=0),
    )(x, Wg, Wu, Wd)


# device time: 28351 ns/iter; 8.7168x vs baseline; 1.1505x over previous
import jax
import jax.numpy as jnp
from jax import lax
from jax.experimental import pallas as pl
from jax.experimental.pallas import tpu as pltpu

N_DEV = 32
CHUNK = 512 // N_DEV
BLK = 128
OWNERS_PER_BLK = BLK // CHUNK


def kernel(x, Wg, Wu, Wd):
    m, d = x.shape

    def body(x_ref, wg_ref, wu_ref, wd_ref, out_ref,
             send_all, rs_recv, ag_recv,
             rs_send_sem, rs_recv_sem, ag_send_sem, ag_recv_sem):
        p = lax.axis_index("i")

        barrier_sem = pltpu.get_barrier_semaphore()
        for o in range(1, N_DEV):
            pl.semaphore_signal(
                barrier_sem, inc=1,
                device_id=(lax.rem(p + o, N_DEV),),
                device_id_type=pl.DeviceIdType.MESH,
            )

        xb = x_ref[...].astype(jnp.bfloat16)
        gate = jnp.dot(xb, wg_ref[...].astype(jnp.bfloat16),
                       preferred_element_type=jnp.float32)
        up = jnp.dot(xb, wu_ref[...].astype(jnp.bfloat16),
                     preferred_element_type=jnp.float32)
        hidden = (gate * (up * jax.nn.sigmoid(up))).astype(jnp.bfloat16)
        wdb = wd_ref[...].astype(jnp.bfloat16)

        pl.semaphore_wait(barrier_sem, N_DEV - 1)

        rs_rdma = None
        for b in range(m // BLK):
            pblk = jnp.dot(hidden[b * BLK:(b + 1) * BLK, :], wdb,
                           preferred_element_type=jnp.float32)
            send_all[pl.ds(b * BLK, BLK), :] = pblk.astype(jnp.bfloat16)
            for t in range(b * OWNERS_PER_BLK, (b + 1) * OWNERS_PER_BLK):
                rs_rdma = pltpu.make_async_remote_copy(
                    src_ref=send_all.at[pl.ds(t * CHUNK, CHUNK)],
                    dst_ref=rs_recv.at[pl.ds(p * CHUNK, CHUNK)],
                    send_sem=rs_send_sem,
                    recv_sem=rs_recv_sem,
                    device_id=(t,),
                    device_id_type=pl.DeviceIdType.MESH,
                )

                @pl.when(t != p)
                def _(rdma=rs_rdma):
                    rdma.start()

        rs_recv[pl.ds(p * CHUNK, CHUNK), :] = send_all[pl.ds(p * CHUNK, CHUNK), :]
        for _ in range(N_DEV - 1):
            rs_rdma.wait_recv()

        red = rs_recv[pl.ds(0, CHUNK), :].astype(jnp.float32)
        for q in range(1, N_DEV):
            red = red + rs_recv[pl.ds(q * CHUNK, CHUNK), :].astype(jnp.float32)

        ag_recv[pl.ds(p * CHUNK, CHUNK), :] = red.astype(jnp.bfloat16)
        for o in range(1, N_DEV):
            t = lax.rem(p + o, N_DEV)
            ag_rdma = pltpu.make_async_remote_copy(
                src_ref=ag_recv.at[pl.ds(p * CHUNK, CHUNK)],
                dst_ref=ag_recv.at[pl.ds(p * CHUNK, CHUNK)],
                send_sem=ag_send_sem,
                recv_sem=ag_recv_sem,
                device_id=(t,),
                device_id_type=pl.DeviceIdType.MESH,
            )
            ag_rdma.start()
        for _ in range(N_DEV - 1):
            ag_rdma.wait_recv()

        out_ref[...] = ag_recv[...].astype(jnp.float32)

        for _ in range(N_DEV - 1):
            rs_rdma.wait_send()
            ag_rdma.wait_send()

    return pl.pallas_call(
        body,
        out_shape=jax.ShapeDtypeStruct((m, d), jnp.float32),
        in_specs=[pl.BlockSpec(memory_space=pltpu.VMEM)] * 4,
        out_specs=pl.BlockSpec(memory_space=pltpu.VMEM),
        scratch_shapes=[
            pltpu.VMEM((m, d), jnp.bfloat16),
            pltpu.VMEM((m, d), jnp.bfloat16),
            pltpu.VMEM((m, d), jnp.bfloat16),
            pltpu.SemaphoreType.DMA,
            pltpu.SemaphoreType.DMA,
            pltpu.SemaphoreType.DMA,
            pltpu.SemaphoreType.DMA,
        ],
        compiler_params=pltpu.CompilerParams(collective_id=0),
    )(x, Wg, Wu, Wd)


# device time: 27757 ns/iter; 8.9033x vs baseline; 1.0214x over previous
import jax
import jax.numpy as jnp
from jax import lax
from jax.experimental import pallas as pl
from jax.experimental.pallas import tpu as pltpu

N_DEV = 32
CHUNK = 512 // N_DEV
BLK = 128
OWNERS_PER_BLK = BLK // CHUNK


def kernel(x, Wg, Wu, Wd):
    m, d = x.shape

    def body(x_ref, wg_ref, wu_ref, wd_ref, out_ref,
             send_all, rs_recv, ag_recv,
             rs_send_sem, rs_recv_sem, ag_send_sem, ag_recv_sem):
        p = lax.axis_index("i")

        barrier_sem = pltpu.get_barrier_semaphore()
        for o in range(1, N_DEV):
            pl.semaphore_signal(
                barrier_sem, inc=1,
                device_id=(lax.rem(p + o, N_DEV),),
                device_id_type=pl.DeviceIdType.MESH,
            )

        xb = x_ref[...].astype(jnp.bfloat16)
        wgb = wg_ref[...].astype(jnp.bfloat16)
        wub = wu_ref[...].astype(jnp.bfloat16)
        wdb = wd_ref[...].astype(jnp.bfloat16)

        rs_rdma = None
        for b in range(m // BLK):
            xblk = xb[b * BLK:(b + 1) * BLK, :]
            gate = jnp.dot(xblk, wgb, preferred_element_type=jnp.float32)
            up = jnp.dot(xblk, wub, preferred_element_type=jnp.float32)
            hblk = (gate * (up * jax.nn.sigmoid(up))).astype(jnp.bfloat16)
            pblk = jnp.dot(hblk, wdb, preferred_element_type=jnp.float32)
            send_all[pl.ds(b * BLK, BLK), :] = pblk.astype(jnp.bfloat16)
            if b == 0:
                pl.semaphore_wait(barrier_sem, N_DEV - 1)
            for t in range(b * OWNERS_PER_BLK, (b + 1) * OWNERS_PER_BLK):
                rs_rdma = pltpu.make_async_remote_copy(
                    src_ref=send_all.at[pl.ds(t * CHUNK, CHUNK)],
                    dst_ref=rs_recv.at[pl.ds(p * CHUNK, CHUNK)],
                    send_sem=rs_send_sem,
                    recv_sem=rs_recv_sem,
                    device_id=(t,),
                    device_id_type=pl.DeviceIdType.MESH,
                )

                @pl.when(t != p)
                def _(rdma=rs_rdma):
                    rdma.start()

        rs_recv[pl.ds(p * CHUNK, CHUNK), :] = send_all[pl.ds(p * CHUNK, CHUNK), :]
        for _ in range(N_DEV - 1):
            rs_rdma.wait_recv()

        terms = [rs_recv[pl.ds(q * CHUNK, CHUNK), :].astype(jnp.float32)
                 for q in range(N_DEV)]
        while len(terms) > 1:
            terms = [terms[i] + terms[i + 1] for i in range(0, len(terms), 2)]
        red = terms[0]

        ag_recv[pl.ds(p * CHUNK, CHUNK), :] = red.astype(jnp.bfloat16)
        for o in range(1, N_DEV):
            t = lax.rem(p + o, N_DEV)
            ag_rdma = pltpu.make_async_remote_copy(
                src_ref=ag_recv.at[pl.ds(p * CHUNK, CHUNK)],
                dst_ref=ag_recv.at[pl.ds(p * CHUNK, CHUNK)],
                send_sem=ag_send_sem,
                recv_sem=ag_recv_sem,
                device_id=(t,),
                device_id_type=pl.DeviceIdType.MESH,
            )
            ag_rdma.start()
        for _ in range(N_DEV - 1):
            ag_rdma.wait_recv()

        out_ref[...] = ag_recv[...].astype(jnp.float32)

        for _ in range(N_DEV - 1):
            rs_rdma.wait_send()
            ag_rdma.wait_send()

    return pl.pallas_call(
        body,
        out_shape=jax.ShapeDtypeStruct((m, d), jnp.float32),
        in_specs=[pl.BlockSpec(memory_space=pltpu.VMEM)] * 4,
        out_specs=pl.BlockSpec(memory_space=pltpu.VMEM),
        scratch_shapes=[
            pltpu.VMEM((m, d), jnp.bfloat16),
            pltpu.VMEM((m, d), jnp.bfloat16),
            pltpu.VMEM((m, d), jnp.bfloat16),
            pltpu.SemaphoreType.DMA,
            pltpu.SemaphoreType.DMA,
            pltpu.SemaphoreType.DMA,
            pltpu.SemaphoreType.DMA,
        ],
        compiler_params=pltpu.CompilerParams(collective_id=0),
    )(x, Wg, Wu, Wd)
